# Initial kernel scaffold; baseline (speedup 1.0000x reference)
import jax
import jax.numpy as jnp
from jax.experimental import pallas as pl


def kernel(x, pos_embed_weight):
    raise NotImplementedError("write your pallas kernel here")


# speedup vs baseline: 5.5870x; 5.5870x over previous
import jax
import jax.numpy as jnp
from jax.experimental import pallas as pl


def _copy_body(w_ref, o_ref):
    w = w_ref[...]
    o_ref[...] = jnp.broadcast_to(w[None], o_ref.shape)


def kernel(x, pos_embed_weight):
    B, S = x.shape
    M, D = pos_embed_weight.shape
    CHUNK = 512
    out = pl.pallas_call(
        _copy_body,
        grid=(S // CHUNK,),
        in_specs=[pl.BlockSpec((CHUNK, D), lambda i: (i, 0))],
        out_specs=pl.BlockSpec((B, CHUNK, D), lambda i: (0, i, 0)),
        out_shape=jax.ShapeDtypeStruct((B, S, D), jnp.float32),
    )(pos_embed_weight)
    return out


# speedup vs baseline: 5.8015x; 1.0384x over previous
import jax
import jax.numpy as jnp
from jax.experimental import pallas as pl


def _copy_body(w_ref, o_ref):
    w = w_ref[...]
    o_ref[...] = jnp.broadcast_to(w[None], o_ref.shape)


def kernel(x, pos_embed_weight):
    B, S = x.shape
    M, D = pos_embed_weight.shape
    CHUNK = 1024
    out = pl.pallas_call(
        _copy_body,
        grid=(S // CHUNK,),
        in_specs=[pl.BlockSpec((CHUNK, D), lambda i: (i, 0))],
        out_specs=pl.BlockSpec((B, CHUNK, D), lambda i: (0, i, 0)),
        out_shape=jax.ShapeDtypeStruct((B, S, D), jnp.float32),
    )(pos_embed_weight)
    return out
